# baseline (speedup 1.0000x reference)
import numpy as np
import jax
import jax.numpy as jnp
from jax.experimental import pallas as pl

_ANCHOR_W = (116.0, 156.0, 373.0)
_ANCHOR_H = (90.0, 198.0, 326.0)
_NG = 64
_NCELL = _NG * _NG
_ATTRS_IN = 87
_ATTRS_OUT = 86
_STRIDE = 512.0 / _NG


def _decode_body(x_ref, o_ref):
    gx = jax.lax.broadcasted_iota(jnp.int32, (_NG, _NG), 1).astype(jnp.float32)
    gy = jax.lax.broadcasted_iota(jnp.int32, (_NG, _NG), 0).astype(jnp.float32)

    rA = jax.lax.broadcasted_iota(jnp.int32, (8, _ATTRS_OUT), 0)
    cA = jax.lax.broadcasted_iota(jnp.int32, (8, _ATTRS_OUT), 1)
    selA = jnp.where((cA < 5) & (rA == cA), 1.0, 0.0).astype(jnp.bfloat16)
    rB = jax.lax.broadcasted_iota(jnp.int32, (_ATTRS_IN, _ATTRS_OUT), 0)
    cB = jax.lax.broadcasted_iota(jnp.int32, (_ATTRS_IN, _ATTRS_OUT), 1)
    selB = jnp.where((cB >= 5) & (rB == cB + 1), 1.0, 0.0).astype(jnp.bfloat16)

    dims = (((0,), (0,)), ((), ()))
    zero = jnp.zeros((_NG, _NG), jnp.float32)

    for a in range(3):
        t = x_ref[0, _ATTRS_IN * a:_ATTRS_IN * (a + 1)]
        s = jax.nn.sigmoid(t)

        px = (s[0] + gx) * _STRIDE
        py = (s[1] + gy) * _STRIDE
        pw = jnp.exp(t[2]) * _ANCHOR_W[a]
        plh = jnp.exp(t[3]) * _ANCHOR_H[a]
        theta = jnp.arctan2(jnp.tanh(t[4]), jnp.tanh(t[5])) * (90.0 / np.pi)

        five = jnp.stack([px, py, pw, plh, theta, zero, zero, zero], axis=0)
        hi = five.astype(jnp.bfloat16)
        lo = (five - hi.astype(jnp.float32)).astype(jnp.bfloat16)
        out = (
            jax.lax.dot_general(s.astype(jnp.bfloat16), selB, dims,
                                preferred_element_type=jnp.float32)
            + jax.lax.dot_general(hi, selA, dims,
                                  preferred_element_type=jnp.float32)
            + jax.lax.dot_general(lo, selA, dims,
                                  preferred_element_type=jnp.float32)
        )
        o_ref[0, _NCELL * a:_NCELL * (a + 1)] = out.reshape(_NCELL, _ATTRS_OUT)


def kernel(x):
    nB = x.shape[0]
    out_shape = jax.ShapeDtypeStruct((nB, 3 * _NCELL, _ATTRS_OUT), jnp.float32)
    return pl.pallas_call(
        _decode_body,
        grid=(nB,),
        in_specs=[
            pl.BlockSpec((1, 3 * _ATTRS_IN, _NG, _NG), lambda b: (b, 0, 0, 0)),
        ],
        out_specs=pl.BlockSpec((1, 3 * _NCELL, _ATTRS_OUT), lambda b: (b, 0, 0)),
        out_shape=out_shape,
    )(x)


# speedup vs baseline: 2.4291x; 2.4291x over previous
import numpy as np
import jax
import jax.numpy as jnp
from jax.experimental import pallas as pl

_ANCHOR_W = (116.0, 156.0, 373.0)
_ANCHOR_H = (90.0, 198.0, 326.0)
_NG = 64
_NCELL = _NG * _NG
_NCH = 261
_ATTRS_IN = 87
_ATTRS_OUT = 86
_STRIDE = 512.0 / _NG
_GY_BLK = 4
_CHUNK = _GY_BLK * _NG
_STEPS_PER_ANCHOR = _NG // _GY_BLK


def _decode_body(x_ref, o_ref):
    g = pl.program_id(0)
    a = g // _STEPS_PER_ANCHOR
    nb = x_ref.shape[0]

    t = x_ref[...].reshape(nb, _CHUNK, _NCH)
    hi = t.astype(jnp.bfloat16)
    lo = (t - hi.astype(jnp.float32)).astype(jnp.bfloat16)

    base = a * _ATTRS_IN
    r = jax.lax.broadcasted_iota(jnp.int32, (_ATTRS_OUT + 2, _NCH), 0)
    c = jax.lax.broadcasted_iota(jnp.int32, (_ATTRS_OUT + 2, _NCH), 1)
    tgt = jnp.where(r < 5, r, jnp.where(r < _ATTRS_OUT, r + 1,
                                        jnp.where(r == _ATTRS_OUT, 5, -1)))
    selC = (c == base + tgt).astype(jnp.bfloat16)
    rD = jax.lax.broadcasted_iota(jnp.int32, (8, _NCH), 0)
    cD = jax.lax.broadcasted_iota(jnp.int32, (8, _NCH), 1)
    selD = ((rD < 6) & (cD == base + rD)).astype(jnp.bfloat16)

    dims = (((1,), (2,)), ((), ()))
    raw = jax.lax.dot_general(selC, hi, dims,
                              preferred_element_type=jnp.float32)
    rawlo = jax.lax.dot_general(selD, lo, dims,
                                preferred_element_type=jnp.float32)

    l_i = jax.lax.broadcasted_iota(jnp.int32, (nb, _CHUNK), 1)
    gx = (l_i % _NG).astype(jnp.float32)
    gy = ((g % _STEPS_PER_ANCHOR) * _GY_BLK + l_i // _NG).astype(jnp.float32)

    aw = jnp.where(a == 0, _ANCHOR_W[0], jnp.where(a == 1, _ANCHOR_W[1], _ANCHOR_W[2]))
    ah = jnp.where(a == 0, _ANCHOR_H[0], jnp.where(a == 1, _ANCHOR_H[1], _ANCHOR_H[2]))

    v = raw[0:6] + rawlo[0:6]
    px = (jax.nn.sigmoid(v[0]) + gx) * _STRIDE
    py = (jax.nn.sigmoid(v[1]) + gy) * _STRIDE
    pw = jnp.exp(v[2]) * aw
    plh = jnp.exp(v[3]) * ah
    theta = jnp.arctan2(jnp.tanh(v[4]), jnp.tanh(raw[_ATTRS_OUT] + rawlo[5])) \
        * (90.0 / np.pi)

    five = jnp.stack([px, py, pw, plh, theta], axis=0)
    rest = jax.nn.sigmoid(raw[5:_ATTRS_OUT])
    o_ref[...] = jnp.concatenate([five, rest], axis=0)


def kernel(x):
    nB = x.shape[0]
    xt = jnp.transpose(x, (0, 2, 3, 1))
    out_shape = jax.ShapeDtypeStruct((_ATTRS_OUT, nB, 3 * _NCELL), jnp.float32)
    out = pl.pallas_call(
        _decode_body,
        grid=(3 * _STEPS_PER_ANCHOR,),
        in_specs=[
            pl.BlockSpec((nB, _GY_BLK, _NG, _NCH),
                         lambda g: (0, g % _STEPS_PER_ANCHOR, 0, 0)),
        ],
        out_specs=pl.BlockSpec((_ATTRS_OUT, nB, _CHUNK), lambda g: (0, 0, g)),
        out_shape=out_shape,
    )(xt)
    return jnp.transpose(out, (1, 2, 0))


# speedup vs baseline: 2.5448x; 1.0476x over previous
import numpy as np
import jax
import jax.numpy as jnp
from jax.experimental import pallas as pl

_ANCHOR_W = (116.0, 156.0, 373.0)
_ANCHOR_H = (90.0, 198.0, 326.0)
_NG = 64
_NCELL = _NG * _NG
_NCH = 261
_ATTRS_IN = 87
_ATTRS_OUT = 86
_STRIDE = 512.0 / _NG
_GY_BLK = 4
_CHUNK = _GY_BLK * _NG
_STEPS_PER_ANCHOR = _NG // _GY_BLK


def _decode_body(x_ref, o_ref):
    g = pl.program_id(0)
    a = g // _STEPS_PER_ANCHOR
    nb = x_ref.shape[0]

    t = x_ref[...].reshape(nb, _CHUNK, _NCH)
    hi = t.astype(jnp.bfloat16)
    _NLO = 192
    lo = (t[:, :, :_NLO] - hi[:, :, :_NLO].astype(jnp.float32)).astype(jnp.bfloat16)

    base = a * _ATTRS_IN
    r = jax.lax.broadcasted_iota(jnp.int32, (_ATTRS_OUT + 2, _NCH), 0)
    c = jax.lax.broadcasted_iota(jnp.int32, (_ATTRS_OUT + 2, _NCH), 1)
    tgt = jnp.where(r < 5, r, jnp.where(r < _ATTRS_OUT, r + 1,
                                        jnp.where(r == _ATTRS_OUT, 5, -1)))
    selC = (c == base + tgt).astype(jnp.bfloat16)
    rD = jax.lax.broadcasted_iota(jnp.int32, (8, _NLO), 0)
    cD = jax.lax.broadcasted_iota(jnp.int32, (8, _NLO), 1)
    selD = ((rD < 6) & (cD == base + rD)).astype(jnp.bfloat16)

    dims = (((1,), (2,)), ((), ()))
    raw = jax.lax.dot_general(selC, hi, dims,
                              preferred_element_type=jnp.float32)
    rawlo = jax.lax.dot_general(selD, lo, dims,
                                preferred_element_type=jnp.float32)

    l_i = jax.lax.broadcasted_iota(jnp.int32, (nb, _CHUNK), 1)
    gx = (l_i % _NG).astype(jnp.float32)
    gy = ((g % _STEPS_PER_ANCHOR) * _GY_BLK + l_i // _NG).astype(jnp.float32)

    aw = jnp.where(a == 0, _ANCHOR_W[0], jnp.where(a == 1, _ANCHOR_W[1], _ANCHOR_W[2]))
    ah = jnp.where(a == 0, _ANCHOR_H[0], jnp.where(a == 1, _ANCHOR_H[1], _ANCHOR_H[2]))

    v = raw[0:6] + rawlo[0:6]
    px = (jax.nn.sigmoid(v[0]) + gx) * _STRIDE
    py = (jax.nn.sigmoid(v[1]) + gy) * _STRIDE
    pw = jnp.exp(v[2]) * aw
    plh = jnp.exp(v[3]) * ah
    theta = jnp.arctan2(jnp.tanh(v[4]), jnp.tanh(raw[_ATTRS_OUT] + rawlo[5])) \
        * (90.0 / np.pi)

    five = jnp.stack([px, py, pw, plh, theta], axis=0)
    rest = jax.nn.sigmoid(raw[5:_ATTRS_OUT])
    o_ref[...] = jnp.concatenate([five, rest], axis=0)


def kernel(x):
    nB = x.shape[0]
    xt = jnp.transpose(x, (0, 2, 3, 1))
    out_shape = jax.ShapeDtypeStruct((_ATTRS_OUT, nB, 3 * _NCELL), jnp.float32)
    out = pl.pallas_call(
        _decode_body,
        grid=(3 * _STEPS_PER_ANCHOR,),
        in_specs=[
            pl.BlockSpec((nB, _GY_BLK, _NG, _NCH),
                         lambda g: (0, g % _STEPS_PER_ANCHOR, 0, 0)),
        ],
        out_specs=pl.BlockSpec((_ATTRS_OUT, nB, _CHUNK), lambda g: (0, 0, g)),
        out_shape=out_shape,
    )(xt)
    return jnp.transpose(out, (1, 2, 0))


# speedup vs baseline: 2.7985x; 1.0997x over previous
import numpy as np
import jax
import jax.numpy as jnp
from jax.experimental import pallas as pl

_ANCHOR_W = (116.0, 156.0, 373.0)
_ANCHOR_H = (90.0, 198.0, 326.0)
_NG = 64
_NCELL = _NG * _NG
_NCH = 261
_ATTRS_IN = 87
_ATTRS_OUT = 86
_STRIDE = 512.0 / _NG
_GY_BLK = 8
_CHUNK = _GY_BLK * _NG
_STEPS_PER_ANCHOR = _NG // _GY_BLK


def _decode_body(x_ref, o_ref):
    g = pl.program_id(0)
    a = g // _STEPS_PER_ANCHOR
    nb = x_ref.shape[0]

    t = x_ref[...].reshape(nb, _CHUNK, _NCH)
    hi = t.astype(jnp.bfloat16)
    _NLO = 192
    lo = (t[:, :, :_NLO] - hi[:, :, :_NLO].astype(jnp.float32)).astype(jnp.bfloat16)

    base = a * _ATTRS_IN
    r = jax.lax.broadcasted_iota(jnp.int32, (_ATTRS_OUT + 2, _NCH), 0)
    c = jax.lax.broadcasted_iota(jnp.int32, (_ATTRS_OUT + 2, _NCH), 1)
    tgt = jnp.where(r < 5, r, jnp.where(r < _ATTRS_OUT, r + 1,
                                        jnp.where(r == _ATTRS_OUT, 5, -1)))
    selC = (c == base + tgt).astype(jnp.bfloat16)
    rD = jax.lax.broadcasted_iota(jnp.int32, (8, _NLO), 0)
    cD = jax.lax.broadcasted_iota(jnp.int32, (8, _NLO), 1)
    selD = ((rD < 6) & (cD == base + rD)).astype(jnp.bfloat16)

    dims = (((1,), (2,)), ((), ()))
    raw = jax.lax.dot_general(selC, hi, dims,
                              preferred_element_type=jnp.float32)
    rawlo = jax.lax.dot_general(selD, lo, dims,
                                preferred_element_type=jnp.float32)

    l_i = jax.lax.broadcasted_iota(jnp.int32, (nb, _CHUNK), 1)
    gx = (l_i % _NG).astype(jnp.float32)
    gy = ((g % _STEPS_PER_ANCHOR) * _GY_BLK + l_i // _NG).astype(jnp.float32)

    aw = jnp.where(a == 0, _ANCHOR_W[0], jnp.where(a == 1, _ANCHOR_W[1], _ANCHOR_W[2]))
    ah = jnp.where(a == 0, _ANCHOR_H[0], jnp.where(a == 1, _ANCHOR_H[1], _ANCHOR_H[2]))

    v = raw[0:6] + rawlo[0:6]
    px = (jax.nn.sigmoid(v[0]) + gx) * _STRIDE
    py = (jax.nn.sigmoid(v[1]) + gy) * _STRIDE
    pw = jnp.exp(v[2]) * aw
    plh = jnp.exp(v[3]) * ah
    theta = jnp.arctan2(jnp.tanh(v[4]), jnp.tanh(raw[_ATTRS_OUT] + rawlo[5])) \
        * (90.0 / np.pi)

    five = jnp.stack([px, py, pw, plh, theta], axis=0)
    rest = jax.nn.sigmoid(raw[5:_ATTRS_OUT])
    o_ref[...] = jnp.concatenate([five, rest], axis=0)


def kernel(x):
    nB = x.shape[0]
    xt = jnp.transpose(x, (0, 2, 3, 1))
    out_shape = jax.ShapeDtypeStruct((_ATTRS_OUT, nB, 3 * _NCELL), jnp.float32)
    out = pl.pallas_call(
        _decode_body,
        grid=(3 * _STEPS_PER_ANCHOR,),
        in_specs=[
            pl.BlockSpec((nB, _GY_BLK, _NG, _NCH),
                         lambda g: (0, g % _STEPS_PER_ANCHOR, 0, 0)),
        ],
        out_specs=pl.BlockSpec((_ATTRS_OUT, nB, _CHUNK), lambda g: (0, 0, g)),
        out_shape=out_shape,
    )(xt)
    return jnp.transpose(out, (1, 2, 0))
